# baseline (speedup 1.0000x reference)
import functools

import jax
import jax.numpy as jnp
from jax import lax
from jax.experimental import pallas as pl
from jax.experimental.pallas import tpu as pltpu
from jax.experimental.pallas import tpu_sc as plsc

N_POINTS = 20000
NUM_CLASSES = 20
C = NUM_CLASSES + 1

NC = 2
NS = 16
L = 16
NW = NC * NS

CHUNK = NW * L
NPAD = ((N_POINTS + CHUNK - 1) // CHUNK) * CHUNK
PT = NPAD // NW
G = PT // L

AUX = 32


@functools.partial(
    pl.kernel,
    out_type=(
        jax.ShapeDtypeStruct((NPAD,), jnp.float32),
        jax.ShapeDtypeStruct((NPAD,), jnp.float32),
        jax.ShapeDtypeStruct((NPAD,), jnp.int32),
        jax.ShapeDtypeStruct((NPAD,), jnp.int32),
    ),
    mesh=plsc.VectorSubcoreMesh(core_axis_name="c", subcore_axis_name="s",
                                num_cores=NC, num_subcores=NS),
    compiler_params=pltpu.CompilerParams(needs_layout_passes=False),
    scratch_types=(
        pltpu.VMEM((PT,), jnp.float32),
        pltpu.VMEM((PT,), jnp.float32),
        pltpu.VMEM((PT * C,), jnp.float32),
        pltpu.VMEM((AUX,), jnp.float32),
        pltpu.VMEM((PT,), jnp.float32),
        pltpu.VMEM((PT,), jnp.float32),
        pltpu.VMEM((PT,), jnp.int32),
        pltpu.VMEM((PT,), jnp.int32),
    ),
)
def _sc_select(xs_h, ys_h, sc_h, aux_h, xo_h, yo_h, co_h, ro_h,
               xs_v, ys_v, sc_v, aux_v, xo_v, yo_v, co_v, ro_v):
    wid = lax.axis_index("s") * NC + lax.axis_index("c")
    base = wid * PT
    pltpu.sync_copy(xs_h.at[pl.ds(base, PT)], xs_v)
    pltpu.sync_copy(ys_h.at[pl.ds(base, PT)], ys_v)
    pltpu.sync_copy(sc_h.at[pl.ds(base * C, PT * C)], sc_v)
    pltpu.sync_copy(aux_h, aux_v)

    lane = lax.broadcasted_iota(jnp.int32, (L,), 0)
    laneC = lane * C
    wv = plsc.load_gather(aux_v, [jnp.full((L,), NUM_CLASSES, jnp.int32)])
    hv = plsc.load_gather(aux_v, [jnp.full((L,), NUM_CLASSES + 1, jnp.int32)])

    def group(g, carry):
        b16 = g * L
        x = xs_v[pl.ds(b16, L)]
        y = ys_v[pl.ds(b16, L)]
        idx0 = laneC + b16 * C
        vals = [plsc.load_gather(sc_v, [idx0 + j]) for j in range(C)]
        m = vals[0]
        am = jnp.zeros((L,), jnp.int32)
        for j in range(1, C):
            gt = vals[j] > m
            m = jnp.where(gt, vals[j], m)
            am = jnp.where(gt, j, am)
        exps = [jnp.exp(v - m) for v in vals]
        while len(exps) > 1:
            nxt = [exps[i] + exps[i + 1] for i in range(0, len(exps) - 1, 2)]
            if len(exps) % 2:
                nxt.append(exps[-1])
            exps = nxt
        maxprob = 1.0 / exps[0]
        amc = jnp.minimum(am, NUM_CLASSES - 1)
        thrv = plsc.load_gather(aux_v, [amc])
        valid = ((x >= 0.0) & (x < wv) & (y >= 0.0) & (y < hv)
                 & (am < NUM_CLASSES))
        res = valid & (maxprob >= thrv)
        xo_v[pl.ds(b16, L)] = jnp.where(res, x, -1.0)
        yo_v[pl.ds(b16, L)] = jnp.where(res, y, -1.0)
        co_v[pl.ds(b16, L)] = jnp.where(res, amc, -1)
        ro_v[pl.ds(b16, L)] = res.astype(jnp.int32)
        return carry

    lax.fori_loop(0, G, group, 0)

    pltpu.sync_copy(xo_v, xo_h.at[pl.ds(base, PT)])
    pltpu.sync_copy(yo_v, yo_h.at[pl.ds(base, PT)])
    pltpu.sync_copy(co_v, co_h.at[pl.ds(base, PT)])
    pltpu.sync_copy(ro_v, ro_h.at[pl.ds(base, PT)])


def kernel(points, scores, score_thr, h, w):
    n = points.shape[0]
    pad = NPAD - n
    xs = jnp.pad(points[:, 0], (0, pad), constant_values=-1.0)
    ys = jnp.pad(points[:, 1], (0, pad), constant_values=-1.0)
    sc = jnp.pad(scores, ((0, pad), (0, 0))).reshape(-1)
    aux = jnp.concatenate([
        score_thr.astype(jnp.float32),
        jnp.asarray(w, jnp.float32)[None],
        jnp.asarray(h, jnp.float32)[None],
        jnp.zeros((AUX - NUM_CLASSES - 2,), jnp.float32),
    ])
    xo, yo, co, ro = _sc_select(xs, ys, sc, aux)
    pred_coords = jnp.stack([xo[:n], yo[:n]], axis=-1)
    pred_classes = co[:n]
    reserved = ro[:n].astype(bool)
    return pred_coords, pred_classes, reserved
